# baseline (speedup 1.0000x reference)
import functools

import jax
import jax.numpy as jnp
from jax import lax
from jax.experimental import pallas as pl
from jax.experimental.pallas import tpu as pltpu
from jax.experimental.pallas import tpu_sc as plsc

B, R, C = 2, 1000, 256
H0 = W0 = 128
H1 = W1 = 64
PH = PW = 7
IMG = 1024.0
N = B * R * PH * PW
NWORK = 32
KBLK = 32
NBLK = 96
SPT = KBLK * NBLK
NT = B * (H0 * W0 + H1 * W1)
NT0 = B * H0 * W0


def _indices_weights(rois):
    boxes = rois.reshape(B * R, 5)
    y1 = boxes[:, 0] / IMG
    x1 = boxes[:, 1] / IMG
    y2 = boxes[:, 2] / IMG
    x2 = boxes[:, 3] / IMG
    h = boxes[:, 2] - boxes[:, 0]
    w = boxes[:, 3] - boxes[:, 1]
    lvl1 = (h > 48.0) | (w > 48.0)
    hm1 = jnp.where(lvl1, float(H1 - 1), float(H0 - 1))
    wm1 = jnp.where(lvl1, float(W1 - 1), float(W0 - 1))
    wrow = jnp.where(lvl1, W1, W0).astype(jnp.int32)
    b_of = jnp.repeat(jnp.arange(B, dtype=jnp.int32), R)
    base = jnp.where(lvl1, NT0 + b_of * H1 * W1, b_of * H0 * W0)

    ar = jnp.arange(PH, dtype=jnp.float32)
    ys = y1[:, None] * hm1[:, None] + ar[None, :] * ((y2 - y1) * hm1 / (PH - 1))[:, None]
    xs = x1[:, None] * wm1[:, None] + ar[None, :] * ((x2 - x1) * wm1 / (PW - 1))[:, None]
    y0f = jnp.floor(ys)
    x0f = jnp.floor(xs)
    wy = ys - y0f
    wx = xs - x0f
    y0 = jnp.clip(y0f, 0.0, hm1[:, None]).astype(jnp.int32)
    y1i = jnp.clip(y0f + 1.0, 0.0, hm1[:, None]).astype(jnp.int32)
    x0 = jnp.clip(x0f, 0.0, wm1[:, None]).astype(jnp.int32)
    x1i = jnp.clip(x0f + 1.0, 0.0, wm1[:, None]).astype(jnp.int32)
    rtop = base[:, None] + y0 * wrow[:, None]
    rbot = base[:, None] + y1i * wrow[:, None]

    i00 = rtop[:, :, None] + x0[:, None, :]
    i10 = rbot[:, :, None] + x0[:, None, :]
    idx2 = jnp.stack([i00, i10], axis=-1).reshape(N * 2)

    wxs = jnp.broadcast_to(wx[:, None, :], (B * R, PH, PW)).reshape(N)
    wys = jnp.broadcast_to(wy[:, :, None], (B * R, PH, PW)).reshape(N)
    w2 = jnp.stack([wxs, wys], axis=-1).reshape(N * 2)
    return idx2, w2


def _build_table(feat0, feat1):
    f0 = feat0.reshape(NT0, C)
    f1 = feat1.reshape(NT - NT0, C)
    blk = 1024
    nb0 = NT0 // blk

    def body(i0_ref, i1_ref, o_ref):
        i = pl.program_id(0)

        @pl.when(i < nb0)
        def _():
            o_ref[...] = i0_ref[...]

        @pl.when(i >= nb0)
        def _():
            o_ref[...] = i1_ref[...]

    table = pl.pallas_call(
        body,
        grid=(NT // blk,),
        in_specs=[
            pl.BlockSpec((blk, C), lambda i: (jnp.minimum(i, nb0 - 1), 0)),
            pl.BlockSpec((blk, C), lambda i: (jnp.maximum(i - nb0, 0), 0)),
        ],
        out_specs=pl.BlockSpec((blk, C), lambda i: (i, 0)),
        out_shape=jax.ShapeDtypeStruct((NT, C), jnp.float32),
    )(f0, f1)

    nb = NT // blk

    def body2(a_ref, b_ref, o_ref):
        i = pl.program_id(0)
        o_ref[:, :C] = a_ref[...]

        @pl.when(i < nb - 1)
        def _():
            o_ref[:, C:] = jnp.concatenate([a_ref[1:], b_ref[:1]], axis=0)

        @pl.when(i == nb - 1)
        def _():
            o_ref[:, C:] = jnp.concatenate([a_ref[1:], a_ref[blk - 1:]], axis=0)

    return pl.pallas_call(
        body2,
        grid=(nb,),
        in_specs=[
            pl.BlockSpec((blk, C), lambda i: (i, 0)),
            pl.BlockSpec((8, C), lambda i: (jnp.minimum((i + 1) * (blk // 8), NT // 8 - 1), 0)),
        ],
        out_specs=pl.BlockSpec((blk, 2 * C), lambda i: (i, 0)),
        out_shape=jax.ShapeDtypeStruct((NT, 2 * C), jnp.float32),
    )(table, table)


def _make_sc_kernel():
    mesh = plsc.VectorSubcoreMesh(core_axis_name="c", subcore_axis_name="s")

    @functools.partial(
        pl.kernel,
        mesh=mesh,
        out_type=jax.ShapeDtypeStruct((N, C), jnp.float32),
        scratch_types=[
            pltpu.VMEM((2 * SPT,), jnp.int32),
            pltpu.VMEM((2 * SPT + 16,), jnp.float32),
            pltpu.VMEM((2, 2 * KBLK, 2 * C), jnp.float32),
            pltpu.VMEM((2, KBLK, C), jnp.float32),
            pltpu.SemaphoreType.DMA,
            pltpu.SemaphoreType.DMA,
            pltpu.SemaphoreType.DMA,
            pltpu.SemaphoreType.DMA,
        ],
    )
    def sck(table_hbm, idx_hbm, w_hbm, out_hbm, idxv, wv, rowsv, outv,
            gsem0, gsem1, osem0, osem1):
        wid = lax.axis_index("c") * 16 + lax.axis_index("s")
        tbase = jnp.minimum(wid * (SPT - 8), N - SPT)
        lane0 = lax.iota(jnp.int32, 16) * 0
        lane1 = lane0 + 1
        gsems = (gsem0, gsem1)
        osems = (osem0, osem1)

        pltpu.sync_copy(idx_hbm.at[pl.ds(tbase * 2, 2 * SPT)], idxv)
        pltpu.sync_copy(w_hbm.at[pl.ds(tbase * 2, 2 * SPT)],
                        wv.at[pl.ds(0, 2 * SPT)])

        def gather(b, slot):
            return pltpu.make_async_copy(
                table_hbm.at[idxv.at[pl.ds(b * 2 * KBLK, 2 * KBLK)]],
                rowsv.at[slot], gsems[slot])

        def outcopy(b, slot):
            return pltpu.make_async_copy(
                outv.at[slot], out_hbm.at[pl.ds(tbase + b * KBLK, KBLK)],
                osems[slot])

        gather(0, 0).start()
        gather(1, 1).start()

        @pl.loop(0, NBLK, step=2)
        def _(blk):
            for s in range(2):
                b = blk + s
                gather(b, s).wait()

                @pl.when(b >= 2)
                def _():
                    outcopy(b, s).wait()

                @pl.loop(0, KBLK, step=2)
                def _(k):
                    for k2 in (k, k + 1):
                        wpair = wv[pl.ds((b * KBLK + k2) * 2, 16)]
                        wxv = wpair.at[lane0].get(mode="promise_in_bounds")
                        wyv = wpair.at[lane1].get(mode="promise_in_bounds")
                        for cc in range(C // 16):
                            sl = pl.ds(cc * 16, 16)
                            sr = pl.ds(C + cc * 16, 16)
                            v00 = rowsv[s, 2 * k2, sl]
                            v01 = rowsv[s, 2 * k2, sr]
                            v10 = rowsv[s, 2 * k2 + 1, sl]
                            v11 = rowsv[s, 2 * k2 + 1, sr]
                            top = v00 + (v01 - v00) * wxv
                            bot = v10 + (v11 - v10) * wxv
                            outv[s, k2, sl] = top + (bot - top) * wyv

                outcopy(b, s).start()

                @pl.when(b + 2 < NBLK)
                def _():
                    gather(b + 2, s).start()

        outcopy(NBLK - 2, 0).wait()
        outcopy(NBLK - 1, 1).wait()

    return sck


_SC_KERNEL_CACHE = []


def _sc_kernel():
    if not _SC_KERNEL_CACHE:
        _SC_KERNEL_CACHE.append(_make_sc_kernel())
    return _SC_KERNEL_CACHE[0]


def kernel(feat0, feat1, rois):
    table = _build_table(feat0, feat1)
    idx4, w2 = _indices_weights(rois)
    out = _sc_kernel()(table, idx4, w2)
    return out.reshape(B, R, PH, PW, C)


# speedup vs baseline: 1.0371x; 1.0371x over previous
import functools

import jax
import jax.numpy as jnp
from jax import lax
from jax.experimental import pallas as pl
from jax.experimental.pallas import tpu as pltpu
from jax.experimental.pallas import tpu_sc as plsc

B, R, C = 2, 1000, 256
H0 = W0 = 128
H1 = W1 = 64
PH = PW = 7
IMG = 1024.0
N = B * R * PH * PW
NWORK = 32
KBLK = 32
NBLK = 96
SPT = KBLK * NBLK
NT = B * (H0 * W0 + H1 * W1)
NT0 = B * H0 * W0


def _indices_weights(rois):
    boxes = rois.reshape(B * R, 5)
    y1 = boxes[:, 0] / IMG
    x1 = boxes[:, 1] / IMG
    y2 = boxes[:, 2] / IMG
    x2 = boxes[:, 3] / IMG
    h = boxes[:, 2] - boxes[:, 0]
    w = boxes[:, 3] - boxes[:, 1]
    lvl1 = (h > 48.0) | (w > 48.0)
    hm1 = jnp.where(lvl1, float(H1 - 1), float(H0 - 1))
    wm1 = jnp.where(lvl1, float(W1 - 1), float(W0 - 1))
    wrow = jnp.where(lvl1, W1, W0).astype(jnp.int32)
    b_of = jnp.repeat(jnp.arange(B, dtype=jnp.int32), R)
    base = jnp.where(lvl1, NT0 + b_of * H1 * W1, b_of * H0 * W0)

    ar = jnp.arange(PH, dtype=jnp.float32)
    ys = y1[:, None] * hm1[:, None] + ar[None, :] * ((y2 - y1) * hm1 / (PH - 1))[:, None]
    xs = x1[:, None] * wm1[:, None] + ar[None, :] * ((x2 - x1) * wm1 / (PW - 1))[:, None]
    y0f = jnp.floor(ys)
    x0f = jnp.floor(xs)
    wy = ys - y0f
    wx = xs - x0f
    y0 = jnp.clip(y0f, 0.0, hm1[:, None]).astype(jnp.int32)
    y1i = jnp.clip(y0f + 1.0, 0.0, hm1[:, None]).astype(jnp.int32)
    x0 = jnp.clip(x0f, 0.0, wm1[:, None]).astype(jnp.int32)
    x1i = jnp.clip(x0f + 1.0, 0.0, wm1[:, None]).astype(jnp.int32)
    rtop = base[:, None] + y0 * wrow[:, None]
    rbot = base[:, None] + y1i * wrow[:, None]

    i00 = rtop[:, :, None] + x0[:, None, :]
    i01 = rtop[:, :, None] + x1i[:, None, :]
    i10 = rbot[:, :, None] + x0[:, None, :]
    i11 = rbot[:, :, None] + x1i[:, None, :]
    idx4 = jnp.stack([i00, i01, i10, i11], axis=-1).reshape(N * 4)

    wxs = jnp.broadcast_to(wx[:, None, :], (B * R, PH, PW)).reshape(N)
    wys = jnp.broadcast_to(wy[:, :, None], (B * R, PH, PW)).reshape(N)
    w2 = jnp.stack([wxs, wys], axis=-1).reshape(N * 2)
    return idx4, w2


def _build_table(feat0, feat1):
    f0 = feat0.reshape(NT0, C)
    f1 = feat1.reshape(NT - NT0, C)
    blk = 1024
    nb0 = NT0 // blk

    def body(i0_ref, i1_ref, o_ref):
        i = pl.program_id(0)

        @pl.when(i < nb0)
        def _():
            o_ref[...] = i0_ref[...]

        @pl.when(i >= nb0)
        def _():
            o_ref[...] = i1_ref[...]

    return pl.pallas_call(
        body,
        grid=(NT // blk,),
        in_specs=[
            pl.BlockSpec((blk, C), lambda i: (jnp.minimum(i, nb0 - 1), 0)),
            pl.BlockSpec((blk, C), lambda i: (jnp.maximum(i - nb0, 0), 0)),
        ],
        out_specs=pl.BlockSpec((blk, C), lambda i: (i, 0)),
        out_shape=jax.ShapeDtypeStruct((NT, C), jnp.float32),
    )(f0, f1)


def _make_sc_kernel():
    mesh = plsc.VectorSubcoreMesh(core_axis_name="c", subcore_axis_name="s")

    @functools.partial(
        pl.kernel,
        mesh=mesh,
        out_type=jax.ShapeDtypeStruct((N, C), jnp.float32),
        scratch_types=[
            pltpu.VMEM((4 * SPT,), jnp.int32),
            pltpu.VMEM((2 * SPT + 16,), jnp.float32),
            pltpu.VMEM((2, 4 * KBLK, C), jnp.float32),
            pltpu.VMEM((2, KBLK, C), jnp.float32),
            pltpu.SemaphoreType.DMA,
            pltpu.SemaphoreType.DMA,
            pltpu.SemaphoreType.DMA,
            pltpu.SemaphoreType.DMA,
        ],
    )
    def sck(table_hbm, idx_hbm, w_hbm, out_hbm, idxv, wv, rowsv, outv,
            gsem0, gsem1, osem0, osem1):
        wid = lax.axis_index("c") * 16 + lax.axis_index("s")
        tbase = jnp.minimum(wid * (SPT - 8), N - SPT)
        lane0 = lax.iota(jnp.int32, 16) * 0
        lane1 = lane0 + 1
        gsems = (gsem0, gsem1)
        osems = (osem0, osem1)

        pltpu.sync_copy(idx_hbm.at[pl.ds(tbase * 4, 4 * SPT)], idxv)
        pltpu.sync_copy(w_hbm.at[pl.ds(tbase * 2, 2 * SPT)],
                        wv.at[pl.ds(0, 2 * SPT)])

        def gather(b, slot):
            return pltpu.make_async_copy(
                table_hbm.at[idxv.at[pl.ds(b * 4 * KBLK, 4 * KBLK)]],
                rowsv.at[slot], gsems[slot])

        def outcopy(b, slot):
            return pltpu.make_async_copy(
                outv.at[slot], out_hbm.at[pl.ds(tbase + b * KBLK, KBLK)],
                osems[slot])

        gather(0, 0).start()
        gather(1, 1).start()

        @pl.loop(0, NBLK, step=2)
        def _(blk):
            for s in range(2):
                b = blk + s
                gather(b, s).wait()

                @pl.when(b >= 2)
                def _():
                    outcopy(b, s).wait()

                @pl.loop(0, KBLK, step=2)
                def _(k):
                    for k2 in (k, k + 1):
                        wpair = wv[pl.ds((b * KBLK + k2) * 2, 16)]
                        wxv = wpair.at[lane0].get(mode="promise_in_bounds")
                        wyv = wpair.at[lane1].get(mode="promise_in_bounds")
                        for cc in range(C // 16):
                            sl = pl.ds(cc * 16, 16)
                            v00 = rowsv[s, 4 * k2, sl]
                            v01 = rowsv[s, 4 * k2 + 1, sl]
                            v10 = rowsv[s, 4 * k2 + 2, sl]
                            v11 = rowsv[s, 4 * k2 + 3, sl]
                            top = v00 + (v01 - v00) * wxv
                            bot = v10 + (v11 - v10) * wxv
                            outv[s, k2, sl] = top + (bot - top) * wyv

                outcopy(b, s).start()

                @pl.when(b + 2 < NBLK)
                def _():
                    gather(b + 2, s).start()

        outcopy(NBLK - 2, 0).wait()
        outcopy(NBLK - 1, 1).wait()

    return sck


_SC_KERNEL_CACHE = []


def _sc_kernel():
    if not _SC_KERNEL_CACHE:
        _SC_KERNEL_CACHE.append(_make_sc_kernel())
    return _SC_KERNEL_CACHE[0]


def kernel(feat0, feat1, rois):
    table = _build_table(feat0, feat1)
    idx4, w2 = _indices_weights(rois)
    out = _sc_kernel()(table, idx4, w2)
    return out.reshape(B, R, PH, PW, C)
